# baseline (speedup 1.0000x reference)
import jax
import jax.numpy as jnp
from jax import lax
from jax.experimental import pallas as pl
from jax.experimental.pallas import tpu as pltpu
from jax.experimental.pallas import tpu_sc as plsc

L_BITS = 20
BATCH = 16384
NUM_CORES = 1
NUM_SUBCORES = 16
NUM_WORKERS = NUM_CORES * NUM_SUBCORES
B_W = BATCH // NUM_WORKERS
CHUNK = 128
N_CHUNKS = B_W // CHUNK
TC_BLOCK = 8192


def _index_body(xt_ref, idx_ref):
    xt = xt_ref[...]
    j = lax.broadcasted_iota(jnp.int32, (L_BITS, 1), 0)
    w = jnp.left_shift(jnp.int32(1), (L_BITS - 1) - j).astype(jnp.float32)
    s = jnp.sum(xt * w, axis=0)
    idx_ref[...] = ((s + jnp.float32((1 << L_BITS) - 1)) * 0.5).astype(jnp.int32)


def _gather_body(idx_hbm, aux_hbm, out_hbm, idx_v, out_v, sem):
    cid = lax.axis_index("c")
    sid = lax.axis_index("s")
    wid = sid * NUM_CORES + cid
    base = wid * B_W

    stages = []
    for c in range(N_CHUNKS):
        stages.append(
            pltpu.async_copy(
                idx_hbm.at[pl.ds(base + c * CHUNK, CHUNK)],
                idx_v.at[pl.ds(c * CHUNK, CHUNK)],
                sem.at[c],
            )
        )
    gathers = []
    for c in range(N_CHUNKS):
        stages[c].wait()
        gathers.append(
            pltpu.async_copy(
                aux_hbm.at[idx_v.at[pl.ds(c * CHUNK, CHUNK)]],
                out_v.at[pl.ds(c * CHUNK, CHUNK)],
                sem.at[N_CHUNKS + c],
            )
        )
    writes = []
    for c in range(N_CHUNKS):
        gathers[c].wait()
        writes.append(
            pltpu.async_copy(
                out_v.at[pl.ds(c * CHUNK, CHUNK)],
                out_hbm.at[pl.ds(base + c * CHUNK, CHUNK)],
                sem.at[2 * N_CHUNKS + c],
            )
        )
    for wr in writes:
        wr.wait()


@jax.jit
def kernel(x, aux):
    indices = pl.pallas_call(
        _index_body,
        grid=(BATCH // TC_BLOCK,),
        in_specs=[pl.BlockSpec((L_BITS, TC_BLOCK), lambda i: (0, i))],
        out_specs=pl.BlockSpec((TC_BLOCK,), lambda i: (i,)),
        out_shape=jax.ShapeDtypeStruct((BATCH,), jnp.int32),
    )(x.T)

    mesh = plsc.VectorSubcoreMesh(core_axis_name="c", subcore_axis_name="s", num_cores=1)
    run = pl.kernel(
        _gather_body,
        out_type=jax.ShapeDtypeStruct((BATCH,), jnp.float32),
        mesh=mesh,
        compiler_params=pltpu.CompilerParams(needs_layout_passes=False),
        scratch_types=[
            pltpu.VMEM((B_W,), jnp.int32),
            pltpu.VMEM((B_W,), jnp.float32),
            pltpu.SemaphoreType.DMA((3 * N_CHUNKS,)),
        ],
    )
    return run(indices, aux)


# speedup vs baseline: 1.0056x; 1.0056x over previous
import jax
import jax.numpy as jnp
from jax import lax
from jax.experimental import pallas as pl
from jax.experimental.pallas import tpu as pltpu
from jax.experimental.pallas import tpu_sc as plsc

L_BITS = 20
BATCH = 16384
NUM_CORES = 1
NUM_SUBCORES = 16
NUM_WORKERS = NUM_CORES * NUM_SUBCORES
B_W = BATCH // NUM_WORKERS
CHUNK = 128
N_CHUNKS = B_W // CHUNK
TC_BLOCK = 16384


def _index_body(xt_ref, idx_ref):
    xt = xt_ref[...]
    j = lax.broadcasted_iota(jnp.int32, (L_BITS, 1), 0)
    w = jnp.left_shift(jnp.int32(1), (L_BITS - 1) - j).astype(jnp.float32)
    s = jnp.sum(xt * w, axis=0)
    idx_ref[...] = ((s + jnp.float32((1 << L_BITS) - 1)) * 0.5).astype(jnp.int32)


def _gather_body(idx_hbm, aux_hbm, out_hbm, idx_v, out_v, sem):
    cid = lax.axis_index("c")
    sid = lax.axis_index("s")
    wid = sid * NUM_CORES + cid
    base = wid * B_W

    stages = []
    for c in range(N_CHUNKS):
        stages.append(
            pltpu.async_copy(
                idx_hbm.at[pl.ds(base + c * CHUNK, CHUNK)],
                idx_v.at[pl.ds(c * CHUNK, CHUNK)],
                sem.at[c],
            )
        )
    gathers = []
    for c in range(N_CHUNKS):
        stages[c].wait()
        gathers.append(
            pltpu.async_copy(
                aux_hbm.at[idx_v.at[pl.ds(c * CHUNK, CHUNK)]],
                out_v.at[pl.ds(c * CHUNK, CHUNK)],
                sem.at[N_CHUNKS + c],
            )
        )
    writes = []
    for c in range(N_CHUNKS):
        gathers[c].wait()
        writes.append(
            pltpu.async_copy(
                out_v.at[pl.ds(c * CHUNK, CHUNK)],
                out_hbm.at[pl.ds(base + c * CHUNK, CHUNK)],
                sem.at[2 * N_CHUNKS + c],
            )
        )
    for wr in writes:
        wr.wait()


@jax.jit
def kernel(x, aux):
    indices = pl.pallas_call(
        _index_body,
        grid=(BATCH // TC_BLOCK,),
        in_specs=[pl.BlockSpec((L_BITS, TC_BLOCK), lambda i: (0, i))],
        out_specs=pl.BlockSpec((TC_BLOCK,), lambda i: (i,)),
        out_shape=jax.ShapeDtypeStruct((BATCH,), jnp.int32),
    )(x.T)

    mesh = plsc.VectorSubcoreMesh(core_axis_name="c", subcore_axis_name="s", num_cores=1)
    run = pl.kernel(
        _gather_body,
        out_type=jax.ShapeDtypeStruct((BATCH,), jnp.float32),
        mesh=mesh,
        compiler_params=pltpu.CompilerParams(needs_layout_passes=False),
        scratch_types=[
            pltpu.VMEM((B_W,), jnp.int32),
            pltpu.VMEM((B_W,), jnp.float32),
            pltpu.SemaphoreType.DMA((3 * N_CHUNKS,)),
        ],
    )
    return run(indices, aux)
